# Initial kernel scaffold; baseline (speedup 1.0000x reference)
import jax
import jax.numpy as jnp
from jax.experimental import pallas as pl


def kernel(xyz, points, W0, b0, g0, bt0, W1, b1, g1, bt1, W2, b2, g2, bt2):
    raise NotImplementedError("write your pallas kernel here")


# speedup vs baseline: 8.2870x; 8.2870x over previous
import functools

import jax
import jax.numpy as jnp
from jax import lax
from jax.experimental import pallas as pl
from jax.experimental.pallas import tpu as pltpu
from jax.experimental.pallas import tpu_sc as plsc

B = 16
N = 4096
S = 512
NS = 32
D_IN = 64
EPS = 1e-5
R2 = float(0.2 ** 2)

NCORES = 2
NSUB = 16
NW = NCORES * NSUB
S_HALF = S // 2


def _fps_body(xyz_ref, idx_ref, nxyz_ref, dist_ref):
    x = xyz_ref[:, 0, :]
    y = xyz_ref[:, 1, :]
    z = xyz_ref[:, 2, :]
    lane_n = lax.broadcasted_iota(jnp.int32, (B, N), 1)
    lane_s = lax.broadcasted_iota(jnp.int32, (B, S), 1)
    dist_ref[...] = jnp.full((B, N), 1e10, jnp.float32)
    idx_ref[...] = jnp.zeros((B, S), jnp.int32)
    nxyz_ref[...] = jnp.zeros((B, 3, S), jnp.float32)

    def body(i, f):
        msel = lane_n == f
        cx = jnp.sum(jnp.where(msel, x, 0.0), axis=1, keepdims=True)
        cy = jnp.sum(jnp.where(msel, y, 0.0), axis=1, keepdims=True)
        cz = jnp.sum(jnp.where(msel, z, 0.0), axis=1, keepdims=True)
        colm = lane_s == i
        idx_ref[...] = jnp.where(colm, jnp.broadcast_to(f, (B, S)), idx_ref[...])
        nxyz_ref[:, 0, :] = jnp.where(colm, jnp.broadcast_to(cx, (B, S)),
                                      nxyz_ref[:, 0, :])
        nxyz_ref[:, 1, :] = jnp.where(colm, jnp.broadcast_to(cy, (B, S)),
                                      nxyz_ref[:, 1, :])
        nxyz_ref[:, 2, :] = jnp.where(colm, jnp.broadcast_to(cz, (B, S)),
                                      nxyz_ref[:, 2, :])
        dx = x - cx
        dy = y - cy
        dz = z - cz
        d = dx * dx + dy * dy + dz * dz
        dist = jnp.minimum(dist_ref[...], d)
        dist_ref[...] = dist
        mx = jnp.max(dist, axis=1, keepdims=True)
        f2 = jnp.min(jnp.where(dist == mx, lane_n, N), axis=1, keepdims=True)
        return f2.astype(jnp.int32)

    lax.fori_loop(0, S, body, jnp.zeros((B, 1), jnp.int32))


def _fps(xyz):
    return pl.pallas_call(
        _fps_body,
        out_shape=(
            jax.ShapeDtypeStruct((B, S), jnp.int32),
            jax.ShapeDtypeStruct((B, 3, S), jnp.float32),
        ),
        scratch_shapes=[
            pltpu.VMEM((B, N), jnp.float32),
        ],
    )(xyz)


_NT = 8


def _pre_body(xyz_ref, pts_ref, w_ref, b_ref, out_ref):
    xp = pts_ref[0]
    xx = xyz_ref[0]
    wx = w_ref[:, 0:3]
    wp = w_ref[:, 3:]
    acc = lax.dot_general(xp, wp, (((1,), (1,)), ((), ())),
                          preferred_element_type=jnp.float32)
    acc = acc + lax.dot_general(xx, wx, (((1,), (1,)), ((), ())),
                                preferred_element_type=jnp.float32)
    acc = acc + b_ref[...]
    out_ref[0] = jnp.concatenate([acc, jnp.zeros((512, 64), jnp.float32)], 1)


def _pretransform(xyz_t, pts_t, w0, b0):
    nt = N // 512
    return pl.pallas_call(
        _pre_body,
        grid=(B, nt),
        in_specs=[
            pl.BlockSpec((1, 512, 3), lambda b, n: (b, n, 0)),
            pl.BlockSpec((1, 512, D_IN), lambda b, n: (b, n, 0)),
            pl.BlockSpec((64, 67), lambda b, n: (0, 0)),
            pl.BlockSpec((64,), lambda b, n: (0,)),
        ],
        out_specs=pl.BlockSpec((1, 512, 128), lambda b, n: (b, n, 0)),
        out_shape=jax.ShapeDtypeStruct((B, N, 128), jnp.float32),
    )(xyz_t, pts_t, w0, b0)


def _q_body(nxyz_ref, w_ref, out_ref):
    wx = w_ref[:, 0:3]
    out_ref[0] = lax.dot_general(nxyz_ref[0], wx, (((1,), (1,)), ((), ())),
                                 preferred_element_type=jnp.float32)


def _q_offsets(new_xyz_t, w0):
    return pl.pallas_call(
        _q_body,
        grid=(B,),
        in_specs=[
            pl.BlockSpec((1, S, 3), lambda b: (b, 0, 0)),
            pl.BlockSpec((64, 67), lambda b: (0, 0)),
        ],
        out_specs=pl.BlockSpec((1, S, 64), lambda b: (b, 0, 0)),
        out_shape=jax.ShapeDtypeStruct((B, S, 64), jnp.float32),
    )(new_xyz_t, w0)


M = B * S * NS
_MT = 1024


def _mlp_body(y_ref, st_ref, g_ref, bt_ref, w_ref, b_ref, out_ref, sout_ref,
              acc_ref, *, cin):
    i = pl.program_id(0)
    ssum = jnp.sum(st_ref[:, 0, 0:cin], axis=0)
    ssq = jnp.sum(st_ref[:, 1, 0:cin], axis=0)
    mean = ssum * (1.0 / M)
    var = ssq * (1.0 / M) - mean * mean
    a = g_ref[...] * lax.rsqrt(var + EPS)
    c = bt_ref[...] - a * mean
    xx = jnp.maximum(y_ref[:, 0:cin] * a + c, 0.0)
    ynext = lax.dot_general(xx, w_ref[...], (((1,), (1,)), ((), ())),
                            preferred_element_type=jnp.float32) + b_ref[...]
    out_ref[...] = ynext

    @pl.when(i == 0)
    def _():
        acc_ref[...] = jnp.zeros_like(acc_ref)

    acc_ref[0, :] += jnp.sum(ynext, axis=0)
    acc_ref[1, :] += jnp.sum(ynext * ynext, axis=0)

    @pl.when(i == pl.num_programs(0) - 1)
    def _():
        sout_ref[0] = acc_ref[...]


def _mlp_layer(y, stats, g, bt, w, b, cin, cout):
    nt = M // _MT
    nw_in = stats.shape[0]
    cin_pad = y.shape[1]
    st_pad = stats.shape[2]
    return pl.pallas_call(
        functools.partial(_mlp_body, cin=cin),
        grid=(nt,),
        in_specs=[
            pl.BlockSpec((_MT, cin_pad), lambda i: (i, 0)),
            pl.BlockSpec((nw_in, 2, st_pad), lambda i: (0, 0, 0)),
            pl.BlockSpec((cin,), lambda i: (0,)),
            pl.BlockSpec((cin,), lambda i: (0,)),
            pl.BlockSpec((cout, cin), lambda i: (0, 0)),
            pl.BlockSpec((cout,), lambda i: (0,)),
        ],
        out_specs=(
            pl.BlockSpec((_MT, cout), lambda i: (i, 0)),
            pl.BlockSpec((1, 2, cout), lambda i: (0, 0, 0)),
        ),
        out_shape=(
            jax.ShapeDtypeStruct((M, cout), jnp.float32),
            jax.ShapeDtypeStruct((1, 2, cout), jnp.float32),
        ),
        scratch_shapes=[pltpu.VMEM((2, cout), jnp.float32)],
    )(y, stats, g, bt, w, b)


_PT = 256


def _pool_body(y_ref, st_ref, g_ref, bt_ref, out_ref):
    ssum = jnp.sum(st_ref[:, 0, :], axis=0)
    ssq = jnp.sum(st_ref[:, 1, :], axis=0)
    mean = ssum * (1.0 / M)
    var = ssq * (1.0 / M) - mean * mean
    a = g_ref[...] * lax.rsqrt(var + EPS)
    c = bt_ref[...] - a * mean
    xx = jnp.maximum(y_ref[...] * a, -c) + c
    xx = xx.reshape(_PT, NS, 128)
    out_ref[...] = jnp.max(xx, axis=1)


def _pool(y3, stats, g, bt):
    nt = (B * S) // _PT
    return pl.pallas_call(
        _pool_body,
        grid=(nt,),
        in_specs=[
            pl.BlockSpec((_PT * NS, 128), lambda i: (i, 0)),
            pl.BlockSpec((1, 2, 128), lambda i: (0, 0, 0)),
            pl.BlockSpec((128,), lambda i: (0,)),
            pl.BlockSpec((128,), lambda i: (0,)),
        ],
        out_specs=pl.BlockSpec((_PT, 128), lambda i: (i, 0)),
        out_shape=jax.ShapeDtypeStruct((B * S, 128), jnp.float32),
    )(y3, stats, g, bt)


CAND = N + 64
INF = 3.0e38


def _rne_bf16(v):
    u = plsc.bitcast(v, jnp.int32)
    u = u + 0x7FFF + jnp.bitwise_and(lax.shift_right_logical(u, 16), 1)
    u = jnp.bitwise_and(u, jnp.int32(-65536))
    return plsc.bitcast(u, jnp.float32)


def _ballq_tec(xyz_hbm, fps_hbm, idx_hbm, x_v, y_v, z_v, fpsrow_v, cxs_v,
               cys_v, czs_v, xb_v, yb_v, zb_v, n2_v, c2s_v, cd_v, ci_v,
               oidx_v):
    cid = lax.axis_index("c")
    sid = lax.axis_index("s")
    wid = sid * NCORES + cid
    b = wid // 2
    h = wid % 2
    pltpu.sync_copy(xyz_hbm.at[b * 3 + 0, 0], x_v)
    pltpu.sync_copy(xyz_hbm.at[b * 3 + 1, 0], y_v)
    pltpu.sync_copy(xyz_hbm.at[b * 3 + 2, 0], z_v)
    pltpu.sync_copy(fps_hbm.at[pl.ds(b * S + h * S_HALF, S_HALF)], fpsrow_v)

    lanes = lax.iota(jnp.int32, 16)

    def prep(j, _):
        sl = pl.ds(j * 16, 16)
        xv, yv, zv = x_v[sl], y_v[sl], z_v[sl]
        xb_v[sl] = _rne_bf16(xv)
        yb_v[sl] = _rne_bf16(yv)
        zb_v[sl] = _rne_bf16(zv)
        n2_v[sl] = xv * xv + yv * yv + zv * zv
        return 0

    lax.fori_loop(0, N // 16, prep, 0)

    def cgather(j, _):
        sl = pl.ds(j * 16, 16)
        iv = fpsrow_v[sl]
        cx = plsc.load_gather(x_v, [iv])
        cy = plsc.load_gather(y_v, [iv])
        cz = plsc.load_gather(z_v, [iv])
        cxs_v[sl] = _rne_bf16(cx)
        cys_v[sl] = _rne_bf16(cy)
        czs_v[sl] = _rne_bf16(cz)
        c2s_v[sl] = cx * cx + cy * cy + cz * cz
        return 0

    lax.fori_loop(0, S_HALF // 16, cgather, 0)

    r2v = jnp.full((16,), R2, jnp.float32)

    def center_body(s, _):
        sv = jnp.full((16,), s, jnp.int32)
        cxv = plsc.load_gather(cxs_v, [sv])
        cyv = plsc.load_gather(cys_v, [sv])
        czv = plsc.load_gather(czs_v, [sv])
        c2v = plsc.load_gather(c2s_v, [sv])

        def chunk(j, cnt):
            sl = pl.ds(j * 16, 16)
            dot = cxv * xb_v[sl] + cyv * yb_v[sl] + czv * zb_v[sl]
            d = (-2.0) * dot + c2v + n2_v[sl]
            m = d <= r2v
            mi = m.astype(jnp.int32)
            c = jnp.sum(mi)

            def store(cnt):
                pos = jnp.full((16,), cnt, jnp.int32) + plsc.cumsum(mi) - mi
                plsc.store_scatter(cd_v, [pos], d, mask=m)
                plsc.store_scatter(ci_v, [pos], lanes + j * 16, mask=m)
                return cnt + c

            return lax.cond(c > 0, store, lambda cnt: cnt, cnt)

        cnt = lax.fori_loop(0, N // 16, chunk, jnp.int32(0))
        padpos = jnp.full((16,), cnt, jnp.int32) + lanes
        plsc.store_scatter(cd_v, [padpos], jnp.full((16,), INF, jnp.float32))
        plsc.store_scatter(ci_v, [padpos], jnp.zeros((16,), jnp.int32))

        def merge(jc, carry):
            k0, v0, k1, v1 = carry
            kc = cd_v[pl.ds(jc * 16, 16)]
            vc = ci_v[pl.ds(jc * 16, 16)]
            kc, vc = plsc.sort_key_val(kc, vc)
            rk = lax.rev(kc, (0,))
            rv = lax.rev(vc, (0,))
            m1 = k1 <= rk
            k1, v1 = plsc.sort_key_val(jnp.where(m1, k1, rk),
                                       jnp.where(m1, v1, rv))
            rk1 = lax.rev(k1, (0,))
            rv1 = lax.rev(v1, (0,))
            m2 = k0 <= rk1
            lo_k = jnp.where(m2, k0, rk1)
            lo_v = jnp.where(m2, v0, rv1)
            hi_k = jnp.where(m2, rk1, k0)
            hi_v = jnp.where(m2, rv1, v0)
            k0, v0 = plsc.sort_key_val(lo_k, lo_v)
            k1, v1 = plsc.sort_key_val(hi_k, hi_v)
            return k0, v0, k1, v1

        init = (jnp.full((16,), INF, jnp.float32), jnp.zeros((16,), jnp.int32),
                jnp.full((16,), INF, jnp.float32), jnp.zeros((16,), jnp.int32))
        nchunks = (cnt + 15) // 16
        k0, v0, k1, v1 = lax.fori_loop(0, nchunks, merge, init)

        fill = jnp.full((16,), jnp.sum(jnp.where(lanes == 0, v0, 0)), jnp.int32)
        v0 = jnp.where(k0 >= INF, fill, v0)
        v1 = jnp.where(k1 >= INF, fill, v1)
        oidx_v[pl.ds(s * NS, 16)] = v0
        oidx_v[pl.ds(s * NS + 16, 16)] = v1
        return 0

    lax.fori_loop(0, S_HALF, center_body, 0)
    pltpu.sync_copy(oidx_v, idx_hbm.at[wid, 0])


def _ball_query(xyz3, fps_flat):
    mesh = plsc.VectorSubcoreMesh(core_axis_name="c", subcore_axis_name="s")
    return pl.kernel(
        _ballq_tec,
        compiler_params=pltpu.CompilerParams(needs_layout_passes=False),
        out_type=jax.ShapeDtypeStruct((NW, 1, S_HALF * NS), jnp.int32),
        mesh=mesh,
        scratch_types=[
            pltpu.VMEM((N,), jnp.float32),
            pltpu.VMEM((N,), jnp.float32),
            pltpu.VMEM((N,), jnp.float32),
            pltpu.VMEM((S_HALF,), jnp.int32),
            pltpu.VMEM((S_HALF,), jnp.float32),
            pltpu.VMEM((S_HALF,), jnp.float32),
            pltpu.VMEM((S_HALF,), jnp.float32),
            pltpu.VMEM((N,), jnp.float32),
            pltpu.VMEM((N,), jnp.float32),
            pltpu.VMEM((N,), jnp.float32),
            pltpu.VMEM((N,), jnp.float32),
            pltpu.VMEM((S_HALF,), jnp.float32),
            pltpu.VMEM((CAND,), jnp.float32),
            pltpu.VMEM((CAND,), jnp.int32),
            pltpu.VMEM((S_HALF * NS,), jnp.int32),
        ],
    )(xyz3, fps_flat)


GW = 128
ROWS_W = S_HALF * NS
NWIN = ROWS_W // GW


def _gather_tec(f_hbm, idx_hbm, q_hbm, y1_hbm, st_hbm, gidx_v, q_v, rows_v,
                acc_v, sem0, sem1):
    cid = lax.axis_index("c")
    sid = lax.axis_index("s")
    wid = sid * NCORES + cid
    b = wid // 2
    h = wid % 2
    base_row = (b * S + h * S_HALF) * NS

    pltpu.sync_copy(idx_hbm.at[wid, 0], gidx_v)
    pltpu.sync_copy(q_hbm.at[pl.ds((b * S + h * S_HALF) * 64, S_HALF * 64)],
                    q_v)

    boff = jnp.full((16,), b * N, jnp.int32)

    def addb(j, _):
        gidx_v[pl.ds(j * 16, 16)] = gidx_v[pl.ds(j * 16, 16)] + boff
        return 0

    lax.fori_loop(0, ROWS_W // 16, addb, 0)

    for i in range(8):
        for t in range(8):
            acc_v[i, pl.ds(t * 16, 16)] = jnp.zeros((16,), jnp.float32)

    def window(w, _):
        copy = pltpu.make_async_copy(f_hbm.at[gidx_v.at[pl.ds(w * GW, GW)]],
                                     rows_v, sem0)
        copy.start()
        copy.wait()

        def row(r, _):
            qi = (w * (GW // NS) + r // NS) * 64
            for t in range(4):
                v = rows_v[r, pl.ds(t * 16, 16)] - q_v[pl.ds(qi + t * 16, 16)]
                rows_v[r, pl.ds(t * 16, 16)] = v
                acc_v[0, pl.ds(t * 16, 16)] += v
                acc_v[1, pl.ds(t * 16, 16)] += v * v
            return 0

        lax.fori_loop(0, GW, row, 0)
        cp2 = pltpu.make_async_copy(
            rows_v, y1_hbm.at[pl.ds(base_row + w * GW, GW)], sem1)
        cp2.start()
        cp2.wait()
        return 0

    lax.fori_loop(0, NWIN, window, 0)
    pltpu.sync_copy(acc_v, st_hbm.at[wid])


def _gather_stage(f2d, idx, qflat):
    mesh = plsc.VectorSubcoreMesh(core_axis_name="c", subcore_axis_name="s")
    return pl.kernel(
        _gather_tec,
        compiler_params=pltpu.CompilerParams(needs_layout_passes=False),
        out_type=(
            jax.ShapeDtypeStruct((M, 128), jnp.float32),
            jax.ShapeDtypeStruct((NW, 8, 128), jnp.float32),
        ),
        mesh=mesh,
        scratch_types=[
            pltpu.VMEM((ROWS_W,), jnp.int32),
            pltpu.VMEM((S_HALF * 64,), jnp.float32),
            pltpu.VMEM((GW, 128), jnp.float32),
            pltpu.VMEM((8, 128), jnp.float32),
            pltpu.SemaphoreType.DMA,
            pltpu.SemaphoreType.DMA,
        ],
    )(f2d, idx, qflat)


def kernel(xyz, points, W0, b0, g0, bt0, W1, b1, g1, bt1, W2, b2, g2, bt2):
    xyz_t = jnp.transpose(xyz, (0, 2, 1))
    pts_t = jnp.transpose(points, (0, 2, 1))

    fps_idx, new_xyz = _fps(xyz)
    f = _pretransform(xyz_t, pts_t, W0, b0)
    q = _q_offsets(jnp.transpose(new_xyz, (0, 2, 1)), W0)
    idx = _ball_query(xyz.reshape(B * 3, 1, N), fps_idx.reshape(B * S))
    y1, st1p = _gather_stage(f.reshape(B * N, 128), idx, q.reshape(-1))
    st1 = st1p[:, 0:2, :]
    y2, st2 = _mlp_layer(y1, st1, g0, bt0, W1, b1, 64, 64)
    y3, st3 = _mlp_layer(y2, st2, g1, bt1, W2, b2, 64, 128)
    pooled = _pool(y3, st3, g2, bt2)
    new_points = jnp.transpose(pooled.reshape(B, S, 128), (0, 2, 1))
    return (new_xyz, new_points, fps_idx)


# speedup vs baseline: 11.0865x; 1.3378x over previous
import functools

import jax
import jax.numpy as jnp
from jax import lax
from jax.experimental import pallas as pl
from jax.experimental.pallas import tpu as pltpu
from jax.experimental.pallas import tpu_sc as plsc

B = 16
N = 4096
S = 512
NS = 32
D_IN = 64
EPS = 1e-5
R2 = float(0.2 ** 2)

NCORES = 2
NSUB = 16
NW = NCORES * NSUB
S_HALF = S // 2


def _fps_body(xyz_ref, idx_ref, nxyz_ref, dist_ref):
    x = xyz_ref[:, 0, :]
    y = xyz_ref[:, 1, :]
    z = xyz_ref[:, 2, :]
    lane_n = lax.broadcasted_iota(jnp.int32, (B, N), 1)
    lane_s = lax.broadcasted_iota(jnp.int32, (B, S), 1)
    dist_ref[...] = jnp.full((B, N), 1e10, jnp.float32)
    idx_ref[...] = jnp.zeros((B, S), jnp.int32)
    nxyz_ref[...] = jnp.zeros((B, 3, S), jnp.float32)

    def body(i, f):
        msel = lane_n == f
        cx = jnp.sum(jnp.where(msel, x, 0.0), axis=1, keepdims=True)
        cy = jnp.sum(jnp.where(msel, y, 0.0), axis=1, keepdims=True)
        cz = jnp.sum(jnp.where(msel, z, 0.0), axis=1, keepdims=True)
        colm = lane_s == i
        idx_ref[...] = jnp.where(colm, jnp.broadcast_to(f, (B, S)), idx_ref[...])
        nxyz_ref[:, 0, :] = jnp.where(colm, jnp.broadcast_to(cx, (B, S)),
                                      nxyz_ref[:, 0, :])
        nxyz_ref[:, 1, :] = jnp.where(colm, jnp.broadcast_to(cy, (B, S)),
                                      nxyz_ref[:, 1, :])
        nxyz_ref[:, 2, :] = jnp.where(colm, jnp.broadcast_to(cz, (B, S)),
                                      nxyz_ref[:, 2, :])
        dx = x - cx
        dy = y - cy
        dz = z - cz
        d = dx * dx + dy * dy + dz * dz
        dist = jnp.minimum(dist_ref[...], d)
        dist_ref[...] = dist
        mx = jnp.max(dist, axis=1, keepdims=True)
        f2 = jnp.min(jnp.where(dist == mx, lane_n, N), axis=1, keepdims=True)
        return f2.astype(jnp.int32)

    lax.fori_loop(0, S, body, jnp.zeros((B, 1), jnp.int32))


def _fps(xyz):
    return pl.pallas_call(
        _fps_body,
        out_shape=(
            jax.ShapeDtypeStruct((B, S), jnp.int32),
            jax.ShapeDtypeStruct((B, 3, S), jnp.float32),
        ),
        scratch_shapes=[
            pltpu.VMEM((B, N), jnp.float32),
        ],
    )(xyz)


_NT = 8


def _pre_body(xyz_ref, pts_ref, w_ref, b_ref, out_ref):
    xp = pts_ref[0]
    xx = xyz_ref[0]
    wx = w_ref[:, 0:3]
    wp = w_ref[:, 3:]
    acc = lax.dot_general(xp, wp, (((1,), (1,)), ((), ())),
                          preferred_element_type=jnp.float32)
    acc = acc + lax.dot_general(xx, wx, (((1,), (1,)), ((), ())),
                                preferred_element_type=jnp.float32)
    acc = acc + b_ref[...]
    out_ref[0] = jnp.concatenate([acc, jnp.zeros((512, 64), jnp.float32)], 1)


def _pretransform(xyz_t, pts_t, w0, b0):
    nt = N // 512
    return pl.pallas_call(
        _pre_body,
        grid=(B, nt),
        in_specs=[
            pl.BlockSpec((1, 512, 3), lambda b, n: (b, n, 0)),
            pl.BlockSpec((1, 512, D_IN), lambda b, n: (b, n, 0)),
            pl.BlockSpec((64, 67), lambda b, n: (0, 0)),
            pl.BlockSpec((64,), lambda b, n: (0,)),
        ],
        out_specs=pl.BlockSpec((1, 512, 128), lambda b, n: (b, n, 0)),
        out_shape=jax.ShapeDtypeStruct((B, N, 128), jnp.float32),
    )(xyz_t, pts_t, w0, b0)


def _q_body(nxyz_ref, w_ref, out_ref):
    wx = w_ref[:, 0:3]
    out_ref[0] = lax.dot_general(nxyz_ref[0], wx, (((1,), (1,)), ((), ())),
                                 preferred_element_type=jnp.float32)


def _q_offsets(new_xyz_t, w0):
    return pl.pallas_call(
        _q_body,
        grid=(B,),
        in_specs=[
            pl.BlockSpec((1, S, 3), lambda b: (b, 0, 0)),
            pl.BlockSpec((64, 67), lambda b: (0, 0)),
        ],
        out_specs=pl.BlockSpec((1, S, 64), lambda b: (b, 0, 0)),
        out_shape=jax.ShapeDtypeStruct((B, S, 64), jnp.float32),
    )(new_xyz_t, w0)


M = B * S * NS
_MT = 1024


def _mlp_body(y_ref, st_ref, g_ref, bt_ref, w_ref, b_ref, out_ref, sout_ref,
              acc_ref, *, cin):
    i = pl.program_id(0)
    ssum = jnp.sum(st_ref[:, 0, 0:cin], axis=0)
    ssq = jnp.sum(st_ref[:, 1, 0:cin], axis=0)
    mean = ssum * (1.0 / M)
    var = ssq * (1.0 / M) - mean * mean
    a = g_ref[...] * lax.rsqrt(var + EPS)
    c = bt_ref[...] - a * mean
    xx = jnp.maximum(y_ref[:, 0:cin] * a + c, 0.0)
    ynext = lax.dot_general(xx, w_ref[...], (((1,), (1,)), ((), ())),
                            preferred_element_type=jnp.float32) + b_ref[...]
    out_ref[...] = ynext

    @pl.when(i == 0)
    def _():
        acc_ref[...] = jnp.zeros_like(acc_ref)

    acc_ref[0, :] += jnp.sum(ynext, axis=0)
    acc_ref[1, :] += jnp.sum(ynext * ynext, axis=0)

    @pl.when(i == pl.num_programs(0) - 1)
    def _():
        sout_ref[0] = acc_ref[...]


def _mlp_layer(y, stats, g, bt, w, b, cin, cout):
    nt = M // _MT
    nw_in = stats.shape[0]
    cin_pad = y.shape[1]
    st_pad = stats.shape[2]
    return pl.pallas_call(
        functools.partial(_mlp_body, cin=cin),
        grid=(nt,),
        in_specs=[
            pl.BlockSpec((_MT, cin_pad), lambda i: (i, 0)),
            pl.BlockSpec((nw_in, 2, st_pad), lambda i: (0, 0, 0)),
            pl.BlockSpec((cin,), lambda i: (0,)),
            pl.BlockSpec((cin,), lambda i: (0,)),
            pl.BlockSpec((cout, cin), lambda i: (0, 0)),
            pl.BlockSpec((cout,), lambda i: (0,)),
        ],
        out_specs=(
            pl.BlockSpec((_MT, cout), lambda i: (i, 0)),
            pl.BlockSpec((1, 2, cout), lambda i: (0, 0, 0)),
        ),
        out_shape=(
            jax.ShapeDtypeStruct((M, cout), jnp.float32),
            jax.ShapeDtypeStruct((1, 2, cout), jnp.float32),
        ),
        scratch_shapes=[pltpu.VMEM((2, cout), jnp.float32)],
    )(y, stats, g, bt, w, b)


_PT = 256


def _pool_body(y_ref, st_ref, g_ref, bt_ref, out_ref):
    ssum = jnp.sum(st_ref[:, 0, :], axis=0)
    ssq = jnp.sum(st_ref[:, 1, :], axis=0)
    mean = ssum * (1.0 / M)
    var = ssq * (1.0 / M) - mean * mean
    a = g_ref[...] * lax.rsqrt(var + EPS)
    c = bt_ref[...] - a * mean
    xx = jnp.maximum(y_ref[...] * a, -c) + c
    xx = xx.reshape(_PT, NS, 128)
    out_ref[...] = jnp.max(xx, axis=1)


def _pool(y3, stats, g, bt):
    nt = (B * S) // _PT
    return pl.pallas_call(
        _pool_body,
        grid=(nt,),
        in_specs=[
            pl.BlockSpec((_PT * NS, 128), lambda i: (i, 0)),
            pl.BlockSpec((1, 2, 128), lambda i: (0, 0, 0)),
            pl.BlockSpec((128,), lambda i: (0,)),
            pl.BlockSpec((128,), lambda i: (0,)),
        ],
        out_specs=pl.BlockSpec((_PT, 128), lambda i: (i, 0)),
        out_shape=jax.ShapeDtypeStruct((B * S, 128), jnp.float32),
    )(y3, stats, g, bt)


CAND = N + 64
INF = 3.0e38


def _rne_bf16(v):
    u = plsc.bitcast(v, jnp.int32)
    u = u + 0x7FFF + jnp.bitwise_and(lax.shift_right_logical(u, 16), 1)
    u = jnp.bitwise_and(u, jnp.int32(-65536))
    return plsc.bitcast(u, jnp.float32)


def _ballq_tec(xyz_hbm, fps_hbm, idx_hbm, x_v, y_v, z_v, fpsrow_v, cxs_v,
               cys_v, czs_v, xb_v, yb_v, zb_v, n2_v, c2s_v, cd_v, ci_v,
               oidx_v):
    cid = lax.axis_index("c")
    sid = lax.axis_index("s")
    wid = sid * NCORES + cid
    b = wid // 2
    h = wid % 2
    pltpu.sync_copy(xyz_hbm.at[b * 3 + 0, 0], x_v)
    pltpu.sync_copy(xyz_hbm.at[b * 3 + 1, 0], y_v)
    pltpu.sync_copy(xyz_hbm.at[b * 3 + 2, 0], z_v)
    pltpu.sync_copy(fps_hbm.at[pl.ds(b * S + h * S_HALF, S_HALF)], fpsrow_v)

    lanes = lax.iota(jnp.int32, 16)

    def prep(j, _):
        sl = pl.ds(j * 16, 16)
        xv, yv, zv = x_v[sl], y_v[sl], z_v[sl]
        xb_v[sl] = _rne_bf16(xv)
        yb_v[sl] = _rne_bf16(yv)
        zb_v[sl] = _rne_bf16(zv)
        n2_v[sl] = xv * xv + yv * yv + zv * zv
        return 0

    lax.fori_loop(0, N // 16, prep, 0)

    def cgather(j, _):
        sl = pl.ds(j * 16, 16)
        iv = fpsrow_v[sl]
        cx = plsc.load_gather(x_v, [iv])
        cy = plsc.load_gather(y_v, [iv])
        cz = plsc.load_gather(z_v, [iv])
        cxs_v[sl] = _rne_bf16(cx)
        cys_v[sl] = _rne_bf16(cy)
        czs_v[sl] = _rne_bf16(cz)
        c2s_v[sl] = cx * cx + cy * cy + cz * cz
        return 0

    lax.fori_loop(0, S_HALF // 16, cgather, 0)

    r2v = jnp.full((16,), R2, jnp.float32)

    def center_body(s, _):
        sv = jnp.full((16,), s, jnp.int32)
        cxv = plsc.load_gather(cxs_v, [sv])
        cyv = plsc.load_gather(cys_v, [sv])
        czv = plsc.load_gather(czs_v, [sv])
        c2v = plsc.load_gather(c2s_v, [sv])

        def chunk4(j4, cnt):
            for u in range(4):
                j = j4 * 4 + u
                sl = pl.ds(j * 16, 16)
                dot = cxv * xb_v[sl] + cyv * yb_v[sl] + czv * zb_v[sl]
                d = (-2.0) * dot + c2v + n2_v[sl]
                m = d <= r2v
                mi = m.astype(jnp.int32)
                pos = cnt + plsc.cumsum(mi) - mi
                plsc.store_scatter(cd_v, [pos], d, mask=m)
                plsc.store_scatter(ci_v, [pos], lanes + j * 16, mask=m)
                cnt = cnt + plsc.all_reduce_population_count(m)
            return cnt

        cntv = lax.fori_loop(0, N // 64, chunk4,
                             jnp.zeros((16,), jnp.int32))
        plsc.store_scatter(cd_v, [cntv + lanes],
                           jnp.full((16,), INF, jnp.float32))
        plsc.store_scatter(ci_v, [cntv + lanes], jnp.zeros((16,), jnp.int32))
        cnt = jnp.sum(jnp.where(lanes == 0, cntv, 0))

        def merge(jc, carry):
            k0, v0, k1, v1 = carry
            kc = cd_v[pl.ds(jc * 16, 16)]
            vc = ci_v[pl.ds(jc * 16, 16)]
            kc, vc = plsc.sort_key_val(kc, vc)
            rk = lax.rev(kc, (0,))
            rv = lax.rev(vc, (0,))
            m1 = k1 <= rk
            k1, v1 = plsc.sort_key_val(jnp.where(m1, k1, rk),
                                       jnp.where(m1, v1, rv))
            rk1 = lax.rev(k1, (0,))
            rv1 = lax.rev(v1, (0,))
            m2 = k0 <= rk1
            lo_k = jnp.where(m2, k0, rk1)
            lo_v = jnp.where(m2, v0, rv1)
            hi_k = jnp.where(m2, rk1, k0)
            hi_v = jnp.where(m2, rv1, v0)
            k0, v0 = plsc.sort_key_val(lo_k, lo_v)
            k1, v1 = plsc.sort_key_val(hi_k, hi_v)
            return k0, v0, k1, v1

        init = (jnp.full((16,), INF, jnp.float32), jnp.zeros((16,), jnp.int32),
                jnp.full((16,), INF, jnp.float32), jnp.zeros((16,), jnp.int32))
        nchunks = (cnt + 15) // 16
        k0, v0, k1, v1 = lax.fori_loop(0, nchunks, merge, init)

        fill = jnp.full((16,), jnp.sum(jnp.where(lanes == 0, v0, 0)), jnp.int32)
        v0 = jnp.where(k0 >= INF, fill, v0)
        v1 = jnp.where(k1 >= INF, fill, v1)
        oidx_v[pl.ds(s * NS, 16)] = v0
        oidx_v[pl.ds(s * NS + 16, 16)] = v1
        return 0

    lax.fori_loop(0, S_HALF, center_body, 0)
    pltpu.sync_copy(oidx_v, idx_hbm.at[wid, 0])


def _ball_query(xyz3, fps_flat):
    mesh = plsc.VectorSubcoreMesh(core_axis_name="c", subcore_axis_name="s")
    return pl.kernel(
        _ballq_tec,
        compiler_params=pltpu.CompilerParams(needs_layout_passes=False),
        out_type=jax.ShapeDtypeStruct((NW, 1, S_HALF * NS), jnp.int32),
        mesh=mesh,
        scratch_types=[
            pltpu.VMEM((N,), jnp.float32),
            pltpu.VMEM((N,), jnp.float32),
            pltpu.VMEM((N,), jnp.float32),
            pltpu.VMEM((S_HALF,), jnp.int32),
            pltpu.VMEM((S_HALF,), jnp.float32),
            pltpu.VMEM((S_HALF,), jnp.float32),
            pltpu.VMEM((S_HALF,), jnp.float32),
            pltpu.VMEM((N,), jnp.float32),
            pltpu.VMEM((N,), jnp.float32),
            pltpu.VMEM((N,), jnp.float32),
            pltpu.VMEM((N,), jnp.float32),
            pltpu.VMEM((S_HALF,), jnp.float32),
            pltpu.VMEM((CAND,), jnp.float32),
            pltpu.VMEM((CAND,), jnp.int32),
            pltpu.VMEM((S_HALF * NS,), jnp.int32),
        ],
    )(xyz3, fps_flat)


GW = 128
ROWS_W = S_HALF * NS
NWIN = ROWS_W // GW


def _gather_tec(f_hbm, idx_hbm, q_hbm, y1_hbm, st_hbm, gidx_v, q_v, rows_v,
                rowsb_v, acc_v, sem0, semb, sem1):
    cid = lax.axis_index("c")
    sid = lax.axis_index("s")
    wid = sid * NCORES + cid
    b = wid // 2
    h = wid % 2
    base_row = (b * S + h * S_HALF) * NS

    pltpu.sync_copy(idx_hbm.at[wid, 0], gidx_v)
    pltpu.sync_copy(q_hbm.at[pl.ds((b * S + h * S_HALF) * 64, S_HALF * 64)],
                    q_v)

    boff = jnp.full((16,), b * N, jnp.int32)

    def addb(j, _):
        gidx_v[pl.ds(j * 16, 16)] = gidx_v[pl.ds(j * 16, 16)] + boff
        return 0

    lax.fori_loop(0, ROWS_W // 16, addb, 0)

    for i in range(8):
        for t in range(8):
            acc_v[i, pl.ds(t * 16, 16)] = jnp.zeros((16,), jnp.float32)

    bufs = (rows_v, rowsb_v)
    sems = (sem0, semb)

    def start_in(w, buf, sem):
        pltpu.make_async_copy(f_hbm.at[gidx_v.at[pl.ds(w * GW, GW)]],
                              buf, sem).start()

    start_in(0, rows_v, sem0)

    def process(w, buf):
        def row4(r4, _):
            for ru in range(4):
                r = r4 * 4 + ru
                qi = (w * (GW // NS) + r // NS) * 64
                for t in range(4):
                    v = (buf[r, pl.ds(t * 16, 16)]
                         - q_v[pl.ds(qi + t * 16, 16)])
                    buf[r, pl.ds(t * 16, 16)] = v
                    acc_v[0, pl.ds(t * 16, 16)] += v
                    acc_v[1, pl.ds(t * 16, 16)] += v * v
            return 0

        lax.fori_loop(0, GW // 4, row4, 0)
        cp2 = pltpu.make_async_copy(
            buf, y1_hbm.at[pl.ds(base_row + w * GW, GW)], sem1)
        cp2.start()
        cp2.wait()

    def window_pair(wp, _):
        for u in range(2):
            w = wp * 2 + u
            cur, csem = bufs[u], sems[u]
            oth, osem = bufs[1 - u], sems[1 - u]

            @pl.when(w + 1 < NWIN)
            def _():
                start_in(w + 1, oth, osem)

            pltpu.make_async_copy(f_hbm.at[gidx_v.at[pl.ds(w * GW, GW)]],
                                  cur, csem).wait()
            process(w, cur)
        return 0

    lax.fori_loop(0, NWIN // 2, window_pair, 0)
    pltpu.sync_copy(acc_v, st_hbm.at[wid])


def _gather_stage(f2d, idx, qflat):
    mesh = plsc.VectorSubcoreMesh(core_axis_name="c", subcore_axis_name="s")
    return pl.kernel(
        _gather_tec,
        compiler_params=pltpu.CompilerParams(needs_layout_passes=False),
        out_type=(
            jax.ShapeDtypeStruct((M, 128), jnp.float32),
            jax.ShapeDtypeStruct((NW, 8, 128), jnp.float32),
        ),
        mesh=mesh,
        scratch_types=[
            pltpu.VMEM((ROWS_W,), jnp.int32),
            pltpu.VMEM((S_HALF * 64,), jnp.float32),
            pltpu.VMEM((GW, 128), jnp.float32),
            pltpu.VMEM((GW, 128), jnp.float32),
            pltpu.VMEM((8, 128), jnp.float32),
            pltpu.SemaphoreType.DMA,
            pltpu.SemaphoreType.DMA,
            pltpu.SemaphoreType.DMA,
        ],
    )(f2d, idx, qflat)


def kernel(xyz, points, W0, b0, g0, bt0, W1, b1, g1, bt1, W2, b2, g2, bt2):
    xyz_t = jnp.transpose(xyz, (0, 2, 1))
    pts_t = jnp.transpose(points, (0, 2, 1))

    fps_idx, new_xyz = _fps(xyz)
    f = _pretransform(xyz_t, pts_t, W0, b0)
    q = _q_offsets(jnp.transpose(new_xyz, (0, 2, 1)), W0)
    idx = _ball_query(xyz.reshape(B * 3, 1, N), fps_idx.reshape(B * S))
    y1, st1p = _gather_stage(f.reshape(B * N, 128), idx, q.reshape(-1))
    st1 = st1p[:, 0:2, :]
    y2, st2 = _mlp_layer(y1, st1, g0, bt0, W1, b1, 64, 64)
    y3, st3 = _mlp_layer(y2, st2, g1, bt1, W2, b2, 64, 128)
    pooled = _pool(y3, st3, g2, bt2)
    new_points = jnp.transpose(pooled.reshape(B, S, 128), (0, 2, 1))
    return (new_xyz, new_points, fps_idx)


# speedup vs baseline: 11.0963x; 1.0009x over previous
import functools

import jax
import jax.numpy as jnp
from jax import lax
from jax.experimental import pallas as pl
from jax.experimental.pallas import tpu as pltpu
from jax.experimental.pallas import tpu_sc as plsc

B = 16
N = 4096
S = 512
NS = 32
D_IN = 64
EPS = 1e-5
R2 = float(0.2 ** 2)

NCORES = 2
NSUB = 16
NW = NCORES * NSUB
S_HALF = S // 2


def _fps_body(xyz_ref, idx_ref, nxyz_ref, dist_ref):
    x = xyz_ref[:, 0, :]
    y = xyz_ref[:, 1, :]
    z = xyz_ref[:, 2, :]
    lane_n = lax.broadcasted_iota(jnp.int32, (B, N), 1)
    lane_s = lax.broadcasted_iota(jnp.int32, (B, S), 1)
    dist_ref[...] = jnp.full((B, N), 1e10, jnp.float32)
    idx_ref[...] = jnp.zeros((B, S), jnp.int32)
    nxyz_ref[...] = jnp.zeros((B, 3, S), jnp.float32)

    def body(i, f):
        msel = lane_n == f
        cx = jnp.sum(jnp.where(msel, x, 0.0), axis=1, keepdims=True)
        cy = jnp.sum(jnp.where(msel, y, 0.0), axis=1, keepdims=True)
        cz = jnp.sum(jnp.where(msel, z, 0.0), axis=1, keepdims=True)
        colm = lane_s == i
        idx_ref[...] = jnp.where(colm, jnp.broadcast_to(f, (B, S)), idx_ref[...])
        nxyz_ref[:, 0, :] = jnp.where(colm, jnp.broadcast_to(cx, (B, S)),
                                      nxyz_ref[:, 0, :])
        nxyz_ref[:, 1, :] = jnp.where(colm, jnp.broadcast_to(cy, (B, S)),
                                      nxyz_ref[:, 1, :])
        nxyz_ref[:, 2, :] = jnp.where(colm, jnp.broadcast_to(cz, (B, S)),
                                      nxyz_ref[:, 2, :])
        dx = x - cx
        dy = y - cy
        dz = z - cz
        d = dx * dx + dy * dy + dz * dz
        dist = jnp.minimum(dist_ref[...], d)
        dist_ref[...] = dist
        mx = jnp.max(dist, axis=1, keepdims=True)
        f2 = jnp.min(jnp.where(dist == mx, lane_n, N), axis=1, keepdims=True)
        return f2.astype(jnp.int32)

    lax.fori_loop(0, S, body, jnp.zeros((B, 1), jnp.int32))


def _fps(xyz):
    return pl.pallas_call(
        _fps_body,
        out_shape=(
            jax.ShapeDtypeStruct((B, S), jnp.int32),
            jax.ShapeDtypeStruct((B, 3, S), jnp.float32),
        ),
        scratch_shapes=[
            pltpu.VMEM((B, N), jnp.float32),
        ],
    )(xyz)


_NT = 8


def _pre_body(xyz_ref, pts_ref, w_ref, b_ref, out_ref):
    xp = pts_ref[0]
    xx = xyz_ref[0]
    wx = w_ref[:, 0:3]
    wp = w_ref[:, 3:]
    acc = lax.dot_general(xp, wp, (((1,), (1,)), ((), ())),
                          preferred_element_type=jnp.float32)
    acc = acc + lax.dot_general(xx, wx, (((1,), (1,)), ((), ())),
                                preferred_element_type=jnp.float32)
    acc = acc + b_ref[...]
    out_ref[0] = jnp.concatenate([acc, jnp.zeros((512, 64), jnp.float32)], 1)


def _pretransform(xyz_t, pts_t, w0, b0):
    nt = N // 512
    return pl.pallas_call(
        _pre_body,
        grid=(B, nt),
        in_specs=[
            pl.BlockSpec((1, 512, 3), lambda b, n: (b, n, 0)),
            pl.BlockSpec((1, 512, D_IN), lambda b, n: (b, n, 0)),
            pl.BlockSpec((64, 67), lambda b, n: (0, 0)),
            pl.BlockSpec((64,), lambda b, n: (0,)),
        ],
        out_specs=pl.BlockSpec((1, 512, 128), lambda b, n: (b, n, 0)),
        out_shape=jax.ShapeDtypeStruct((B, N, 128), jnp.float32),
    )(xyz_t, pts_t, w0, b0)


def _q_body(nxyz_ref, w_ref, out_ref):
    wx = w_ref[:, 0:3]
    out_ref[0] = lax.dot_general(nxyz_ref[0], wx, (((1,), (1,)), ((), ())),
                                 preferred_element_type=jnp.float32)


def _q_offsets(new_xyz_t, w0):
    return pl.pallas_call(
        _q_body,
        grid=(B,),
        in_specs=[
            pl.BlockSpec((1, S, 3), lambda b: (b, 0, 0)),
            pl.BlockSpec((64, 67), lambda b: (0, 0)),
        ],
        out_specs=pl.BlockSpec((1, S, 64), lambda b: (b, 0, 0)),
        out_shape=jax.ShapeDtypeStruct((B, S, 64), jnp.float32),
    )(new_xyz_t, w0)


M = B * S * NS
_MT = 1024


def _mlp_body(y_ref, st_ref, g_ref, bt_ref, w_ref, b_ref, out_ref, sout_ref,
              acc_ref, *, cin):
    i = pl.program_id(0)
    ssum = jnp.sum(st_ref[:, 0, 0:cin], axis=0)
    ssq = jnp.sum(st_ref[:, 1, 0:cin], axis=0)
    mean = ssum * (1.0 / M)
    var = ssq * (1.0 / M) - mean * mean
    a = g_ref[...] * lax.rsqrt(var + EPS)
    c = bt_ref[...] - a * mean
    xx = jnp.maximum(y_ref[:, 0:cin] * a + c, 0.0)
    ynext = lax.dot_general(xx, w_ref[...], (((1,), (1,)), ((), ())),
                            preferred_element_type=jnp.float32) + b_ref[...]
    out_ref[...] = ynext

    @pl.when(i == 0)
    def _():
        acc_ref[...] = jnp.zeros_like(acc_ref)

    acc_ref[0, :] += jnp.sum(ynext, axis=0)
    acc_ref[1, :] += jnp.sum(ynext * ynext, axis=0)

    @pl.when(i == pl.num_programs(0) - 1)
    def _():
        sout_ref[0] = acc_ref[...]


def _mlp_layer(y, stats, g, bt, w, b, cin, cout):
    nt = M // _MT
    nw_in = stats.shape[0]
    cin_pad = y.shape[1]
    st_pad = stats.shape[2]
    return pl.pallas_call(
        functools.partial(_mlp_body, cin=cin),
        grid=(nt,),
        in_specs=[
            pl.BlockSpec((_MT, cin_pad), lambda i: (i, 0)),
            pl.BlockSpec((nw_in, 2, st_pad), lambda i: (0, 0, 0)),
            pl.BlockSpec((cin,), lambda i: (0,)),
            pl.BlockSpec((cin,), lambda i: (0,)),
            pl.BlockSpec((cout, cin), lambda i: (0, 0)),
            pl.BlockSpec((cout,), lambda i: (0,)),
        ],
        out_specs=(
            pl.BlockSpec((_MT, cout), lambda i: (i, 0)),
            pl.BlockSpec((1, 2, cout), lambda i: (0, 0, 0)),
        ),
        out_shape=(
            jax.ShapeDtypeStruct((M, cout), jnp.float32),
            jax.ShapeDtypeStruct((1, 2, cout), jnp.float32),
        ),
        scratch_shapes=[pltpu.VMEM((2, cout), jnp.float32)],
    )(y, stats, g, bt, w, b)


_PT = 256


def _pool_body(y_ref, st_ref, g_ref, bt_ref, out_ref):
    ssum = jnp.sum(st_ref[:, 0, :], axis=0)
    ssq = jnp.sum(st_ref[:, 1, :], axis=0)
    mean = ssum * (1.0 / M)
    var = ssq * (1.0 / M) - mean * mean
    a = g_ref[...] * lax.rsqrt(var + EPS)
    c = bt_ref[...] - a * mean
    xx = jnp.maximum(y_ref[...] * a, -c) + c
    xx = xx.reshape(_PT, NS, 128)
    out_ref[...] = jnp.max(xx, axis=1)


def _pool(y3, stats, g, bt):
    nt = (B * S) // _PT
    return pl.pallas_call(
        _pool_body,
        grid=(nt,),
        in_specs=[
            pl.BlockSpec((_PT * NS, 128), lambda i: (i, 0)),
            pl.BlockSpec((1, 2, 128), lambda i: (0, 0, 0)),
            pl.BlockSpec((128,), lambda i: (0,)),
            pl.BlockSpec((128,), lambda i: (0,)),
        ],
        out_specs=pl.BlockSpec((_PT, 128), lambda i: (i, 0)),
        out_shape=jax.ShapeDtypeStruct((B * S, 128), jnp.float32),
    )(y3, stats, g, bt)


CAND = N + 64
INF = 3.0e38


def _rne_bf16(v):
    u = plsc.bitcast(v, jnp.int32)
    u = u + 0x7FFF + jnp.bitwise_and(lax.shift_right_logical(u, 16), 1)
    u = jnp.bitwise_and(u, jnp.int32(-65536))
    return plsc.bitcast(u, jnp.float32)


def _ballq_tec(xyz_hbm, fps_hbm, idx_hbm, x_v, y_v, z_v, fpsrow_v, cxs_v,
               cys_v, czs_v, xb_v, yb_v, zb_v, n2_v, c2s_v, cd_v, ci_v,
               oidx_v):
    cid = lax.axis_index("c")
    sid = lax.axis_index("s")
    wid = sid * NCORES + cid
    b = wid // 2
    h = wid % 2
    pltpu.sync_copy(xyz_hbm.at[b * 3 + 0, 0], x_v)
    pltpu.sync_copy(xyz_hbm.at[b * 3 + 1, 0], y_v)
    pltpu.sync_copy(xyz_hbm.at[b * 3 + 2, 0], z_v)
    pltpu.sync_copy(fps_hbm.at[pl.ds(b * S + h * S_HALF, S_HALF)], fpsrow_v)

    lanes = lax.iota(jnp.int32, 16)

    def prep(j, _):
        sl = pl.ds(j * 16, 16)
        xv, yv, zv = x_v[sl], y_v[sl], z_v[sl]
        xb_v[sl] = _rne_bf16(xv)
        yb_v[sl] = _rne_bf16(yv)
        zb_v[sl] = _rne_bf16(zv)
        n2_v[sl] = xv * xv + yv * yv + zv * zv
        return 0

    lax.fori_loop(0, N // 16, prep, 0)

    def cgather(j, _):
        sl = pl.ds(j * 16, 16)
        iv = fpsrow_v[sl]
        cx = plsc.load_gather(x_v, [iv])
        cy = plsc.load_gather(y_v, [iv])
        cz = plsc.load_gather(z_v, [iv])
        cxs_v[sl] = _rne_bf16(cx)
        cys_v[sl] = _rne_bf16(cy)
        czs_v[sl] = _rne_bf16(cz)
        c2s_v[sl] = cx * cx + cy * cy + cz * cz
        return 0

    lax.fori_loop(0, S_HALF // 16, cgather, 0)

    r2v = jnp.full((16,), R2, jnp.float32)

    def center_body(s, _):
        sv = jnp.full((16,), s, jnp.int32)
        cxv = plsc.load_gather(cxs_v, [sv])
        cyv = plsc.load_gather(cys_v, [sv])
        czv = plsc.load_gather(czs_v, [sv])
        c2v = plsc.load_gather(c2s_v, [sv])

        def chunk4(j4, cnt):
            for u in range(8):
                j = j4 * 8 + u
                sl = pl.ds(j * 16, 16)
                dot = cxv * xb_v[sl] + cyv * yb_v[sl] + czv * zb_v[sl]
                d = (-2.0) * dot + c2v + n2_v[sl]
                m = d <= r2v
                mi = m.astype(jnp.int32)
                pos = cnt + plsc.cumsum(mi) - mi
                plsc.store_scatter(cd_v, [pos], d, mask=m)
                plsc.store_scatter(ci_v, [pos], lanes + j * 16, mask=m)
                cnt = cnt + plsc.all_reduce_population_count(m)
            return cnt

        cntv = lax.fori_loop(0, N // 128, chunk4,
                             jnp.zeros((16,), jnp.int32))
        plsc.store_scatter(cd_v, [cntv + lanes],
                           jnp.full((16,), INF, jnp.float32))
        plsc.store_scatter(ci_v, [cntv + lanes], jnp.zeros((16,), jnp.int32))
        cnt = jnp.sum(jnp.where(lanes == 0, cntv, 0))

        def merge(jc, carry):
            k0, v0, k1, v1 = carry
            kc = cd_v[pl.ds(jc * 16, 16)]
            vc = ci_v[pl.ds(jc * 16, 16)]
            kc, vc = plsc.sort_key_val(kc, vc)
            rk = lax.rev(kc, (0,))
            rv = lax.rev(vc, (0,))
            m1 = k1 <= rk
            k1, v1 = plsc.sort_key_val(jnp.where(m1, k1, rk),
                                       jnp.where(m1, v1, rv))
            rk1 = lax.rev(k1, (0,))
            rv1 = lax.rev(v1, (0,))
            m2 = k0 <= rk1
            lo_k = jnp.where(m2, k0, rk1)
            lo_v = jnp.where(m2, v0, rv1)
            hi_k = jnp.where(m2, rk1, k0)
            hi_v = jnp.where(m2, rv1, v0)
            k0, v0 = plsc.sort_key_val(lo_k, lo_v)
            k1, v1 = plsc.sort_key_val(hi_k, hi_v)
            return k0, v0, k1, v1

        init = (jnp.full((16,), INF, jnp.float32), jnp.zeros((16,), jnp.int32),
                jnp.full((16,), INF, jnp.float32), jnp.zeros((16,), jnp.int32))
        nchunks = (cnt + 15) // 16
        k0, v0, k1, v1 = lax.fori_loop(0, nchunks, merge, init)

        fill = jnp.full((16,), jnp.sum(jnp.where(lanes == 0, v0, 0)), jnp.int32)
        v0 = jnp.where(k0 >= INF, fill, v0)
        v1 = jnp.where(k1 >= INF, fill, v1)
        oidx_v[pl.ds(s * NS, 16)] = v0
        oidx_v[pl.ds(s * NS + 16, 16)] = v1
        return 0

    lax.fori_loop(0, S_HALF, center_body, 0)
    pltpu.sync_copy(oidx_v, idx_hbm.at[wid, 0])


def _ball_query(xyz3, fps_flat):
    mesh = plsc.VectorSubcoreMesh(core_axis_name="c", subcore_axis_name="s")
    return pl.kernel(
        _ballq_tec,
        compiler_params=pltpu.CompilerParams(needs_layout_passes=False),
        out_type=jax.ShapeDtypeStruct((NW, 1, S_HALF * NS), jnp.int32),
        mesh=mesh,
        scratch_types=[
            pltpu.VMEM((N,), jnp.float32),
            pltpu.VMEM((N,), jnp.float32),
            pltpu.VMEM((N,), jnp.float32),
            pltpu.VMEM((S_HALF,), jnp.int32),
            pltpu.VMEM((S_HALF,), jnp.float32),
            pltpu.VMEM((S_HALF,), jnp.float32),
            pltpu.VMEM((S_HALF,), jnp.float32),
            pltpu.VMEM((N,), jnp.float32),
            pltpu.VMEM((N,), jnp.float32),
            pltpu.VMEM((N,), jnp.float32),
            pltpu.VMEM((N,), jnp.float32),
            pltpu.VMEM((S_HALF,), jnp.float32),
            pltpu.VMEM((CAND,), jnp.float32),
            pltpu.VMEM((CAND,), jnp.int32),
            pltpu.VMEM((S_HALF * NS,), jnp.int32),
        ],
    )(xyz3, fps_flat)


GW = 128
ROWS_W = S_HALF * NS
NWIN = ROWS_W // GW


def _gather_tec(f_hbm, idx_hbm, q_hbm, y1_hbm, st_hbm, gidx_v, q_v, rows_v,
                rowsb_v, acc_v, sem0, semb, sem1):
    cid = lax.axis_index("c")
    sid = lax.axis_index("s")
    wid = sid * NCORES + cid
    b = wid // 2
    h = wid % 2
    base_row = (b * S + h * S_HALF) * NS

    pltpu.sync_copy(idx_hbm.at[wid, 0], gidx_v)
    pltpu.sync_copy(q_hbm.at[pl.ds((b * S + h * S_HALF) * 64, S_HALF * 64)],
                    q_v)

    boff = jnp.full((16,), b * N, jnp.int32)

    def addb(j, _):
        gidx_v[pl.ds(j * 16, 16)] = gidx_v[pl.ds(j * 16, 16)] + boff
        return 0

    lax.fori_loop(0, ROWS_W // 16, addb, 0)

    for i in range(8):
        for t in range(8):
            acc_v[i, pl.ds(t * 16, 16)] = jnp.zeros((16,), jnp.float32)

    bufs = (rows_v, rowsb_v)
    sems = (sem0, semb)

    def start_in(w, buf, sem):
        pltpu.make_async_copy(f_hbm.at[gidx_v.at[pl.ds(w * GW, GW)]],
                              buf, sem).start()

    start_in(0, rows_v, sem0)

    def process(w, buf):
        def row4(r4, _):
            for ru in range(8):
                r = r4 * 8 + ru
                qi = (w * (GW // NS) + r // NS) * 64
                for t in range(4):
                    v = (buf[r, pl.ds(t * 16, 16)]
                         - q_v[pl.ds(qi + t * 16, 16)])
                    buf[r, pl.ds(t * 16, 16)] = v
                    acc_v[0, pl.ds(t * 16, 16)] += v
                    acc_v[1, pl.ds(t * 16, 16)] += v * v
            return 0

        lax.fori_loop(0, GW // 8, row4, 0)
        cp2 = pltpu.make_async_copy(
            buf, y1_hbm.at[pl.ds(base_row + w * GW, GW)], sem1)
        cp2.start()
        cp2.wait()

    def window_pair(wp, _):
        for u in range(2):
            w = wp * 2 + u
            cur, csem = bufs[u], sems[u]
            oth, osem = bufs[1 - u], sems[1 - u]

            @pl.when(w + 1 < NWIN)
            def _():
                start_in(w + 1, oth, osem)

            pltpu.make_async_copy(f_hbm.at[gidx_v.at[pl.ds(w * GW, GW)]],
                                  cur, csem).wait()
            process(w, cur)
        return 0

    lax.fori_loop(0, NWIN // 2, window_pair, 0)
    pltpu.sync_copy(acc_v, st_hbm.at[wid])


def _gather_stage(f2d, idx, qflat):
    mesh = plsc.VectorSubcoreMesh(core_axis_name="c", subcore_axis_name="s")
    return pl.kernel(
        _gather_tec,
        compiler_params=pltpu.CompilerParams(needs_layout_passes=False),
        out_type=(
            jax.ShapeDtypeStruct((M, 128), jnp.float32),
            jax.ShapeDtypeStruct((NW, 8, 128), jnp.float32),
        ),
        mesh=mesh,
        scratch_types=[
            pltpu.VMEM((ROWS_W,), jnp.int32),
            pltpu.VMEM((S_HALF * 64,), jnp.float32),
            pltpu.VMEM((GW, 128), jnp.float32),
            pltpu.VMEM((GW, 128), jnp.float32),
            pltpu.VMEM((8, 128), jnp.float32),
            pltpu.SemaphoreType.DMA,
            pltpu.SemaphoreType.DMA,
            pltpu.SemaphoreType.DMA,
        ],
    )(f2d, idx, qflat)


def kernel(xyz, points, W0, b0, g0, bt0, W1, b1, g1, bt1, W2, b2, g2, bt2):
    xyz_t = jnp.transpose(xyz, (0, 2, 1))
    pts_t = jnp.transpose(points, (0, 2, 1))

    fps_idx, new_xyz = _fps(xyz)
    f = _pretransform(xyz_t, pts_t, W0, b0)
    q = _q_offsets(jnp.transpose(new_xyz, (0, 2, 1)), W0)
    idx = _ball_query(xyz.reshape(B * 3, 1, N), fps_idx.reshape(B * S))
    y1, st1p = _gather_stage(f.reshape(B * N, 128), idx, q.reshape(-1))
    st1 = st1p[:, 0:2, :]
    y2, st2 = _mlp_layer(y1, st1, g0, bt0, W1, b1, 64, 64)
    y3, st3 = _mlp_layer(y2, st2, g1, bt1, W2, b2, 64, 128)
    pooled = _pool(y3, st3, g2, bt2)
    new_points = jnp.transpose(pooled.reshape(B, S, 128), (0, 2, 1))
    return (new_xyz, new_points, fps_idx)
